# baseline (speedup 1.0000x reference)
import functools
import jax
import jax.numpy as jnp
from jax import lax
from jax.experimental import pallas as pl
from jax.experimental.pallas import tpu as pltpu
from jax.experimental.pallas import tpu_sc as plsc

NUM_CLASSES = 100000
DIM = 64
BATCH = 16384
NC = 2
NS = 16
NW = NC * NS
ROWS_PER_W = BATCH // NW
CHUNK = 128
NCHUNK = ROWS_PER_W // CHUNK
LANES = 16


def _sc_body(y_hbm, df_hbm, ct_hbm, out_hbm, idx_v, df_v, ct_v, acc_v, sem):
    wid = lax.axis_index("s") * NC + lax.axis_index("c")
    base = wid * NCHUNK
    pltpu.sync_copy(y_hbm.at[pl.ds(base, NCHUNK)], idx_v)
    copies = [
        pltpu.async_copy(ct_hbm.at[idx_v.at[j]], ct_v.at[j], sem)
        for j in range(NCHUNK)
    ]
    copies.append(pltpu.async_copy(df_hbm.at[pl.ds(base, NCHUNK)], df_v, sem))
    for c in copies:
        c.wait()

    def row_body(i, acc):
        for j in range(NCHUNK):
            for c in range(DIM // LANES):
                d = (df_v[j, i, pl.ds(c * LANES, LANES)]
                     - ct_v[j, i, pl.ds(c * LANES, LANES)])
                acc = acc + d * d
        return acc

    acc_v[...] = lax.fori_loop(0, CHUNK, row_body,
                               jnp.zeros((LANES,), jnp.float32))
    pltpu.sync_copy(acc_v, out_hbm.at[wid])


_sc_call = pl.kernel(
    _sc_body,
    out_type=jax.ShapeDtypeStruct((NW, LANES), jnp.float32),
    mesh=plsc.VectorSubcoreMesh(core_axis_name="c", subcore_axis_name="s"),
    compiler_params=pltpu.CompilerParams(use_tc_tiling_on_sc=False),
    scratch_types=[
        pltpu.VMEM((NCHUNK, CHUNK), jnp.int32),
        pltpu.VMEM((NCHUNK, CHUNK, DIM), jnp.float32),
        pltpu.VMEM((NCHUNK, CHUNK, DIM), jnp.float32),
        pltpu.VMEM((LANES,), jnp.float32),
        pltpu.SemaphoreType.DMA,
    ],
)


@jax.jit
def kernel(y, deep_feat, centers):
    y2 = y.reshape(NW * NCHUNK, CHUNK).astype(jnp.int32)
    df3 = deep_feat.reshape(NW * NCHUNK, CHUNK, DIM)
    partials = _sc_call(y2, df3, centers)
    return 0.5 * jnp.sqrt(jnp.sum(partials)) / BATCH
